# baseline (speedup 1.0000x reference)
import math

import jax
import jax.numpy as jnp
import numpy as np
from jax.experimental import pallas as pl
from jax.experimental.pallas import tpu as pltpu

B = 4
S = 1024
H = 16
KVH = 4
G = H // KVH
D = 128
WINDOW = 512
SCALE = 0.08838834764831845
LOG2E = 1.4426950408889634


def _slopes(n):
    def pow2(n):
        start = 2 ** (-(2 ** (-(math.log2(n) - 3))))
        return [start * start ** i for i in range(n)]
    if math.log2(n).is_integer():
        return pow2(n)
    closest = 2 ** math.floor(math.log2(n))
    return pow2(closest) + _slopes(2 * closest)[0::2][: n - closest]


BLOCKS = (
    (0, 256, 0, 256),
    (256, 256, 0, 512),
    (512, 256, 0, 768),
    (768, 256, 256, 768),
)


def _attn_kernel(slopes_ref, q_ref, k_ref, v_ref, o_ref):
    h = pl.program_id(1)

    k_scaled = k_ref[0, :, :] * jnp.float32(SCALE * LOG2E)

    for q_base, bq, start, ks in BLOCKS:
        kspan = k_scaled[start:start + ks, :]
        vspan = v_ref[0, start:start + ks, :]
        i_idx = q_base + jax.lax.broadcasted_iota(jnp.int32, (bq, ks), 0)
        j_idx = start + jax.lax.broadcasted_iota(jnp.int32, (bq, ks), 1)
        valid = (j_idx <= i_idx) & (j_idx >= i_idx - WINDOW)
        delta_masked = jnp.where(
            valid, (j_idx - i_idx).astype(jnp.float32), jnp.float32(-1e30))

        qall = jnp.concatenate(
            [q_ref[0, q_base:q_base + bq, g * D:(g + 1) * D]
             for g in range(G)], axis=0)
        s = jax.lax.dot_general(
            qall, kspan, (((1,), (1,)), ((), ())),
            preferred_element_type=jnp.float32,
        )
        bias = jnp.concatenate(
            [slopes_ref[h, g] * delta_masked for g in range(G)], axis=0)
        p = jnp.exp2(s + bias)
        l = jnp.sum(p, axis=1, keepdims=True)
        oall = jax.lax.dot_general(
            p, vspan, (((1,), (0,)), ((), ())),
            preferred_element_type=jnp.float32,
        ) * (1.0 / l)
        for g in range(G):
            o_ref[0, q_base:q_base + bq, g * D:(g + 1) * D] = \
                oall[g * bq:(g + 1) * bq, :]


def kernel(q, k, v):
    qh = q.reshape(B, S, H * D)
    kh = k.reshape(B, S, KVH * D)
    vh = v.reshape(B, S, KVH * D)
    slopes = jnp.asarray(
        (np.array(_slopes(H), dtype=np.float64) * LOG2E)
        .astype(np.float32).reshape(KVH, G))

    out = pl.pallas_call(
        _attn_kernel,
        grid=(B, KVH),
        in_specs=[
            pl.BlockSpec(memory_space=pltpu.SMEM),
            pl.BlockSpec((1, S, G * D), lambda b, h: (b, 0, h)),
            pl.BlockSpec((1, S, D), lambda b, h: (b, 0, h)),
            pl.BlockSpec((1, S, D), lambda b, h: (b, 0, h)),
        ],
        out_specs=pl.BlockSpec((1, S, G * D), lambda b, h: (b, 0, h)),
        out_shape=jax.ShapeDtypeStruct((B, S, H * D), jnp.float32),
        compiler_params=pltpu.CompilerParams(
            dimension_semantics=("parallel", "parallel")),
    )(slopes, qh, kh, vh)
    return out.reshape(B * S, H * D)
